# baseline (speedup 1.0000x reference)
import jax
import jax.numpy as jnp
from jax.experimental import pallas as pl

_EPS = 1e-5


def _dot(a, b):
    return jnp.dot(a.astype(jnp.bfloat16), b.astype(jnp.bfloat16),
                   preferred_element_type=jnp.float32)


def _act(x, m, den, g, b, a):
    z = (x - m) / den * g + b
    return jnp.where(z > 0, z, a * z)


def _bn_vecs(rows, n, hs, ws, g, b, alpha):
    c = rows.shape[1]
    t = jnp.transpose(rows.reshape(n, hs, ws, c), (0, 3, 1, 2))
    mean = jnp.mean(t, axis=(0, 2, 3))
    var = jnp.var(t, axis=(0, 2, 3))
    den = jnp.sqrt(var + _EPS)
    return mean[None, :], den[None, :], g[None, :], b[None, :], alpha[None, :]


def _mm_body(x_ref, m_ref, d_ref, g_ref, b_ref, a_ref, w_ref, y_ref):
    act = _act(x_ref[...], m_ref[...], d_ref[...], g_ref[...], b_ref[...],
               a_ref[...])
    y_ref[...] = _dot(act, w_ref[...])


def _mm(x, vecs, w):
    m, cin = x.shape
    cout = w.shape[1]
    bm = m if m <= 3136 else 3136
    grid = (m // bm,)
    vspec = pl.BlockSpec((1, cin), lambda i: (0, 0))
    return pl.pallas_call(
        _mm_body,
        grid=grid,
        in_specs=[pl.BlockSpec((bm, cin), lambda i: (i, 0)),
                  vspec, vspec, vspec, vspec, vspec,
                  pl.BlockSpec((cin, cout), lambda i: (0, 0))],
        out_specs=pl.BlockSpec((bm, cout), lambda i: (i, 0)),
        out_shape=jax.ShapeDtypeStruct((m, cout), jnp.float32),
    )(x, *vecs, w)


def _conv3_body(x_ref, m_ref, d_ref, g_ref, b_ref, a_ref, w_ref, y_ref):
    _, h, w_sp, c = x_ref.shape
    act = _act(x_ref[...], m_ref[...], d_ref[...], g_ref[...], b_ref[...],
               a_ref[...]).astype(jnp.bfloat16)
    p = jnp.pad(act, ((0, 0), (1, 1), (1, 1), (0, 0)))
    sls = [p[:, dy:dy + h, dx:dx + w_sp, :].reshape(h * w_sp, c)
           for dy in range(3) for dx in range(3)]
    im = jnp.concatenate(sls, axis=1)
    y = jnp.dot(im, w_ref[...].astype(jnp.bfloat16),
                preferred_element_type=jnp.float32)
    y_ref[...] = y[None]


def _conv3(x_nhwc, vecs, w):
    n, h, w_sp, c = x_nhwc.shape
    cout = w.shape[3]
    m = n * h * w_sp
    vspec = pl.BlockSpec((1, c), lambda i: (0, 0))
    y = pl.pallas_call(
        _conv3_body,
        grid=(n,),
        in_specs=[pl.BlockSpec((1, h, w_sp, c), lambda i: (i, 0, 0, 0)),
                  vspec, vspec, vspec, vspec, vspec,
                  pl.BlockSpec((9 * c, cout), lambda i: (0, 0))],
        out_specs=pl.BlockSpec((1, h * w_sp, cout), lambda i: (i, 0, 0)),
        out_shape=jax.ShapeDtypeStruct((n, h * w_sp, cout), jnp.float32),
    )(x_nhwc, *vecs, w.reshape(9 * c, cout))
    return y.reshape(m, cout)


def _stem_pool_body(x_ref, m_ref, d_ref, g_ref, b_ref, a_ref, y_ref):
    n, h, w_sp, c = x_ref.shape
    ho, wo = h // 2, w_sp // 2
    act = _act(x_ref[...], m_ref[...], d_ref[...], g_ref[...], b_ref[...],
               a_ref[...])
    p = jnp.pad(act, ((0, 0), (0, 2), (0, 2), (0, 0)))
    pr = p.reshape(n, ho + 1, 2, w_sp + 2, c)
    rows = [pr[:, :ho, 0], pr[:, :ho, 1], pr[:, 1:ho + 1, 0]]
    taps = []
    for r in rows:
        rc = r.reshape(n, ho, wo + 1, 2, c)
        taps.append([rc[:, :, :wo, 0], rc[:, :, :wo, 1],
                     rc[:, :, 1:wo + 1, 0]])
    cols = [taps[0][j] + taps[1][j] + taps[2][j] for j in range(3)]
    s = cols[0] + cols[1] + cols[2]
    y_ref[...] = (s / 9.0).reshape(n * ho * wo, c)


def _stem_pool(x_nhwc, vecs):
    n, h, w_sp, c = x_nhwc.shape
    m = n * (h // 2) * (w_sp // 2)
    return pl.pallas_call(
        _stem_pool_body,
        out_shape=jax.ShapeDtypeStruct((m, c), jnp.float32),
    )(x_nhwc, *vecs)


def _trans_body(x_ref, m_ref, d_ref, g_ref, b_ref, a_ref, w_ref, y_ref):
    _, h, w_sp, c = x_ref.shape
    cout = w_ref.shape[1]
    ho, wo = h // 2, w_sp // 2
    act = _act(x_ref[...], m_ref[...], d_ref[...], g_ref[...], b_ref[...],
               a_ref[...])
    y = _dot(act.reshape(h * w_sp, c), w_ref[...])
    pr = y.reshape(ho, 2, w_sp, cout)
    r0 = pr[:, 0].reshape(ho, wo, 2, cout)
    r1 = pr[:, 1].reshape(ho, wo, 2, cout)
    s = ((r0[:, :, 0] + r0[:, :, 1]) + r1[:, :, 0]) + r1[:, :, 1]
    y_ref[...] = (s / 4.0).reshape(ho * wo, cout)[None]


def _trans(x_nhwc, vecs, w):
    n, h, w_sp, c = x_nhwc.shape
    cout = w.shape[1]
    ho, wo = h // 2, w_sp // 2
    m = n * ho * wo
    vspec = pl.BlockSpec((1, c), lambda i: (0, 0))
    y = pl.pallas_call(
        _trans_body,
        grid=(n,),
        in_specs=[pl.BlockSpec((1, h, w_sp, c), lambda i: (i, 0, 0, 0)),
                  vspec, vspec, vspec, vspec, vspec,
                  pl.BlockSpec((c, cout), lambda i: (0, 0))],
        out_specs=pl.BlockSpec((1, ho * wo, cout), lambda i: (i, 0, 0)),
        out_shape=jax.ShapeDtypeStruct((n, ho * wo, cout), jnp.float32),
    )(x_nhwc, *vecs, w)
    return y.reshape(m, cout)


def _head_body(x_ref, m_ref, d_ref, g_ref, b_ref, a_ref, w_ref, og_ref,
               ob_ref, opr_ref, y_ref):
    n = 4
    mrows, c = x_ref.shape
    sp = mrows // n
    act = _act(x_ref[...], m_ref[...], d_ref[...], g_ref[...], b_ref[...],
               a_ref[...])
    pooled = jnp.sum(act.reshape(n, sp, c), axis=1) / float(sp)
    y = _dot(pooled, w_ref[...])
    mu = jnp.sum(y, axis=0, keepdims=True) / float(n)
    dv = y - mu
    var = jnp.sum(dv * dv, axis=0, keepdims=True) / float(n)
    yn = (y - mu) / jnp.sqrt(var + _EPS) * og_ref[...] + ob_ref[...]
    y_ref[...] = jnp.where(yn > 0, yn, opr_ref[...] * yn)


def _head(x2d, vecs, w, og, ob, opr):
    n = 4
    cout = w.shape[1]
    return pl.pallas_call(
        _head_body,
        out_shape=jax.ShapeDtypeStruct((n, cout), jnp.float32),
    )(x2d, *vecs, w, og[None], ob[None], opr[None])


_BLOCKS = (6, 12, 24, 16)


def kernel(x, params):
    p = params
    n = x.shape[0]
    xh = jnp.transpose(x, (0, 2, 3, 1))
    xp = jnp.pad(xh, ((0, 0), (2, 3), (2, 3), (0, 0)))
    taps = [xp[:, ky:ky + 223:2, kx:kx + 223:2, :]
            for ky in range(7) for kx in range(7)]
    im = jnp.stack(taps, axis=3).reshape(n * 112 * 112, 147)
    w0 = p['conv0'].reshape(147, 64)
    zero = jnp.zeros((147,), jnp.float32)
    one = jnp.ones((147,), jnp.float32)
    ident = (zero[None], one[None], one[None], zero[None], one[None])
    h0 = _mm(im, ident, w0)

    vecs = _bn_vecs(h0, n, 112, 112, p['norm0_g'], p['norm0_b'], p['relu0'])
    h = _stem_pool(h0.reshape(n, 112, 112, 64), vecs)

    hs = 56
    stm, std_ = _bn_vecs(h, n, hs, hs, jnp.zeros(64), jnp.zeros(64),
                         jnp.zeros(64))[:2]
    means, dens = [stm], [std_]
    for bi in range(len(_BLOCKS)):
        for lp in p['blocks'][bi]:
            cin = h.shape[1]
            mcat = jnp.concatenate(means, axis=1)
            dcat = jnp.concatenate(dens, axis=1)
            vecs = (mcat, dcat, lp['bn1_g'][None], lp['bn1_b'][None],
                    lp['pr1'][None])
            b = _mm(h, vecs, lp['conv1'].reshape(cin, -1))
            vecs2 = _bn_vecs(b, n, hs, hs, lp['bn2_g'], lp['bn2_b'],
                             lp['pr2'])
            o = _conv3(b.reshape(n, hs, hs, -1), vecs2, lp['conv2'])
            h = jnp.concatenate([h, o], axis=1)
            om, od = _bn_vecs(o, n, hs, hs, jnp.zeros(32), jnp.zeros(32),
                              jnp.zeros(32))[:2]
            means.append(om)
            dens.append(od)
        if bi != len(_BLOCKS) - 1:
            tp = p['trans'][bi]
            cin = h.shape[1]
            mcat = jnp.concatenate(means, axis=1)
            dcat = jnp.concatenate(dens, axis=1)
            vecs = (mcat, dcat, tp['bn_g'][None], tp['bn_b'][None],
                    tp['pr'][None])
            h = _trans(h.reshape(n, hs, hs, cin), vecs,
                       tp['conv'].reshape(cin, -1))
            hs //= 2
            cm, cd = _bn_vecs(h, n, hs, hs, jnp.zeros(h.shape[1]),
                              jnp.zeros(h.shape[1]),
                              jnp.zeros(h.shape[1]))[:2]
            means, dens = [cm], [cd]

    mcat = jnp.concatenate(means, axis=1)
    dcat = jnp.concatenate(dens, axis=1)
    vecs = (mcat, dcat, p['final_g'][None], p['final_b'][None],
            p['final_pr'][None])
    y = _head(h, vecs, p['linear'], p['obn_g'], p['obn_b'], p['opr'])
    return y
